# baseline (speedup 1.0000x reference)
import functools

import jax
import jax.numpy as jnp
from jax import lax
from jax.experimental import pallas as pl
from jax.experimental.pallas import tpu as pltpu
from jax.experimental.pallas import tpu_sc as plsc

E = 8
K = 2
NEG_INF = float("-inf")

M_BLK = 128
NT = 2048 * K // M_BLK + E
HB = 1536
NC, NS = 2, 16
NW = NC * NS


def _router_kernel(x_ref, gw_ref, i0_ref, i1_ref, g0_ref, g1_ref, aux_ref):
    x = x_ref[...]
    gw = gw_ref[...]
    logits = jax.lax.dot_general(
        x, gw, (((1,), (0,)), ((), ())), preferred_element_type=jnp.float32)
    T = logits.shape[0]
    ii = jax.lax.broadcasted_iota(jnp.int32, (T, E), 1)
    v0 = jnp.max(logits, axis=1, keepdims=True)
    i0 = jnp.min(jnp.where(logits == v0, ii, E), axis=1, keepdims=True)
    masked = jnp.where(ii == i0, NEG_INF, logits)
    v1 = jnp.max(masked, axis=1, keepdims=True)
    i1 = jnp.min(jnp.where(masked == v1, ii, E), axis=1, keepdims=True)
    s = jnp.exp(v1 - v0)
    g0 = 1.0 / (1.0 + s)
    g1 = s / (1.0 + s)
    i0_ref[...] = i0
    i1_ref[...] = i1
    g0_ref[...] = g0
    g1_ref[...] = g1
    gates = jnp.where(ii == i0, g0, 0.0) + jnp.where(ii == i1, g1, 0.0)
    m = jnp.max(logits, axis=1, keepdims=True)
    p = jnp.exp(logits - m)
    p = p / jnp.sum(p, axis=1, keepdims=True)
    f = jnp.mean(gates, axis=0, keepdims=True)
    P = jnp.mean(p, axis=0, keepdims=True)
    aux_ref[0, 0] = E * jnp.sum(f * P)


def _gmm_kernel(te_ref, na_ref, xs_ref, wg_ref, wu_ref, wd_ref,
                os_ref, acc_ref, wgb_ref, wub_ref, wdb_ref, key_ref):
    h = pl.program_id(0)
    t = pl.program_id(1)
    n_hb = pl.num_programs(0)

    @pl.when((h == 0) & (t == 0))
    def _():
        key_ref[0] = -1

    @pl.when(t < na_ref[0])
    def _():
        key = te_ref[t] * 256 + h

        @pl.when(key != key_ref[0])
        def _():
            wgb_ref[...] = wg_ref[0].astype(jnp.bfloat16)
            wub_ref[...] = wu_ref[0].astype(jnp.bfloat16)
            wdb_ref[...] = wd_ref[0].astype(jnp.bfloat16)
            key_ref[0] = key

        xb = xs_ref[0].astype(jnp.bfloat16)
        wgb = wgb_ref[...]
        wub = wub_ref[...]
        wdb = wdb_ref[...]
        g = jax.lax.dot_general(
            xb, wgb, (((1,), (0,)), ((), ())), preferred_element_type=jnp.float32)
        u = jax.lax.dot_general(
            xb, wub, (((1,), (0,)), ((), ())), preferred_element_type=jnp.float32)
        hid = ((g * jax.lax.logistic(g)) * u).astype(jnp.bfloat16)
        part = jax.lax.dot_general(
            hid, wdb, (((1,), (0,)), ((), ())), preferred_element_type=jnp.float32)

        @pl.when(h == 0)
        def _():
            acc_ref[t] = part

        @pl.when(h > 0)
        def _():
            acc_ref[t] += part

        @pl.when(h == n_hb - 1)
        def _():
            os_ref[0] = acc_ref[t]


def _make_dispatch(T, D, NTM):
    t_per_w = T // NW
    a_per_w = K * T // NW
    mesh = plsc.VectorSubcoreMesh(
        core_axis_name="c", subcore_axis_name="s", num_cores=NC, num_subcores=NS)

    @functools.partial(
        pl.kernel, mesh=mesh,
        out_type=jax.ShapeDtypeStruct((NTM, D), jnp.float32),
        scratch_types=[
            pltpu.VMEM((t_per_w,), jnp.int32),
            pltpu.VMEM((t_per_w, D), jnp.float32),
            pltpu.SemaphoreType.DMA,
        ],
    )
    def dispatch(x_hbm, pos0_hbm, pos1_hbm, xs_hbm, idx_v, rows_v, sem):
        wid = lax.axis_index("s") * NC + lax.axis_index("c")
        base = wid * t_per_w
        pltpu.sync_copy(x_hbm.at[pl.ds(base, t_per_w)], rows_v)
        pltpu.sync_copy(pos0_hbm.at[pl.ds(base, t_per_w)], idx_v)
        pltpu.async_copy(rows_v, xs_hbm.at[idx_v], sem).wait()
        pltpu.sync_copy(pos1_hbm.at[pl.ds(base, t_per_w)], idx_v)
        pltpu.async_copy(rows_v, xs_hbm.at[idx_v], sem).wait()

    return dispatch


def _make_combine(T, D, NTM):
    t_per_w = T // NW
    mesh = plsc.VectorSubcoreMesh(
        core_axis_name="c", subcore_axis_name="s", num_cores=NC, num_subcores=NS)

    @functools.partial(
        pl.kernel, mesh=mesh,
        out_type=[
            jax.ShapeDtypeStruct((T, D), jnp.float32),
            jax.ShapeDtypeStruct((T, D), jnp.float32),
        ],
        scratch_types=[
            pltpu.VMEM((t_per_w,), jnp.int32),
            pltpu.VMEM((t_per_w, D), jnp.float32),
            pltpu.SemaphoreType.DMA,
        ],
    )
    def combine(os_hbm, pos0_hbm, pos1_hbm, y0_hbm, y1_hbm, idx_v, rows_v, sem):
        wid = lax.axis_index("s") * NC + lax.axis_index("c")
        base = wid * t_per_w
        pltpu.sync_copy(pos0_hbm.at[pl.ds(base, t_per_w)], idx_v)
        pltpu.async_copy(os_hbm.at[idx_v], rows_v, sem).wait()
        pltpu.sync_copy(rows_v, y0_hbm.at[pl.ds(base, t_per_w)])
        pltpu.sync_copy(pos1_hbm.at[pl.ds(base, t_per_w)], idx_v)
        pltpu.async_copy(os_hbm.at[idx_v], rows_v, sem).wait()
        pltpu.sync_copy(rows_v, y1_hbm.at[pl.ds(base, t_per_w)])

    return combine


def _add_kernel(a_ref, b_ref, ga_ref, gb_ref, o_ref):
    o_ref[...] = a_ref[...] * ga_ref[...] + b_ref[...] * gb_ref[...]


@functools.partial(jax.jit, static_argnames=())
def kernel(x, gate_w, wg, wu, wd):
    B, S, D = x.shape
    H = wg.shape[2]
    T = B * S
    NTM = NT * M_BLK
    x_flat = x.reshape(T, D)

    i0, i1, g0, g1, aux = pl.pallas_call(
        _router_kernel,
        out_shape=(
            jax.ShapeDtypeStruct((T, 1), jnp.int32),
            jax.ShapeDtypeStruct((T, 1), jnp.int32),
            jax.ShapeDtypeStruct((T, 1), jnp.float32),
            jax.ShapeDtypeStruct((T, 1), jnp.float32),
            jax.ShapeDtypeStruct((1, 1), jnp.float32),
        ),
        in_specs=[
            pl.BlockSpec((T, D), lambda: (0, 0)),
            pl.BlockSpec((D, E), lambda: (0, 0)),
        ],
        out_specs=(
            pl.BlockSpec((T, 1), lambda: (0, 0)),
            pl.BlockSpec((T, 1), lambda: (0, 0)),
            pl.BlockSpec((T, 1), lambda: (0, 0)),
            pl.BlockSpec((T, 1), lambda: (0, 0)),
            pl.BlockSpec((1, 1), lambda: (0, 0), memory_space=pltpu.SMEM),
        ),
    )(x_flat, gate_w)
    aux_loss = aux[0, 0]

    e_iota = jnp.arange(E, dtype=jnp.int32)[None, :]
    oh0 = (i0 == e_iota).astype(jnp.int32)
    oh1 = (i1 == e_iota).astype(jnp.int32)
    both = oh0 + oh1
    counts = jnp.sum(both, axis=0)
    excl = jnp.cumsum(both, axis=0) - both
    rank0 = jnp.sum(excl * oh0, axis=1)
    rank1 = jnp.sum((excl + oh0) * oh1, axis=1)
    psize = ((counts + M_BLK - 1) // M_BLK) * M_BLK
    pstart = jnp.concatenate([jnp.zeros((1,), jnp.int32),
                              jnp.cumsum(psize)[:-1].astype(jnp.int32)])
    pos0 = jnp.sum(oh0 * pstart[None, :], axis=1) + rank0
    pos1 = jnp.sum(oh1 * pstart[None, :], axis=1) + rank1
    total = pstart[-1] + psize[-1]
    n_active = (total // M_BLK).astype(jnp.int32)[None]
    tile_start = jnp.arange(NT, dtype=jnp.int32) * M_BLK
    te = jnp.clip(jnp.sum(tile_start[:, None] >= pstart[None, :], axis=1) - 1,
                  0, E - 1).astype(jnp.int32)
    xs = _make_dispatch(T, D, NTM)(x_flat, pos0, pos1)

    n_hb = H // HB
    os_sorted = pl.pallas_call(
        _gmm_kernel,
        grid_spec=pltpu.PrefetchScalarGridSpec(
            num_scalar_prefetch=2,
            grid=(n_hb, NT),
            in_specs=[
                pl.BlockSpec((1, M_BLK, D), lambda h, t, te_r, na_r: (t, 0, 0)),
                pl.BlockSpec((1, D, HB), lambda h, t, te_r, na_r: (te_r[t], 0, h)),
                pl.BlockSpec((1, D, HB), lambda h, t, te_r, na_r: (te_r[t], 0, h)),
                pl.BlockSpec((1, HB, D), lambda h, t, te_r, na_r: (te_r[t], h, 0)),
            ],
            out_specs=pl.BlockSpec((1, M_BLK, D), lambda h, t, te_r, na_r: (t, 0, 0)),
            scratch_shapes=[pltpu.VMEM((NT, M_BLK, D), jnp.float32),
                            pltpu.VMEM((D, HB), jnp.bfloat16),
                            pltpu.VMEM((D, HB), jnp.bfloat16),
                            pltpu.VMEM((HB, D), jnp.bfloat16),
                            pltpu.SMEM((1,), jnp.int32)],
        ),
        out_shape=jax.ShapeDtypeStruct((NT, M_BLK, D), jnp.float32),
    )(te, n_active, xs.reshape(NT, M_BLK, D), wg, wu, wd)

    y0, y1 = _make_combine(T, D, NTM)(
        os_sorted.reshape(NTM, D), pos0, pos1)

    out = pl.pallas_call(
        _add_kernel,
        out_shape=jax.ShapeDtypeStruct((T, D), jnp.float32),
        in_specs=[pl.BlockSpec((T, D), lambda: (0, 0)),
                  pl.BlockSpec((T, D), lambda: (0, 0)),
                  pl.BlockSpec((T, 1), lambda: (0, 0)),
                  pl.BlockSpec((T, 1), lambda: (0, 0))],
        out_specs=pl.BlockSpec((T, D), lambda: (0, 0)),
    )(y0, y1, g0, g1)

    return out.reshape(B, S, D), aux_loss


# speedup vs baseline: 1.0896x; 1.0896x over previous
import functools

import jax
import jax.numpy as jnp
from jax import lax
from jax.experimental import pallas as pl
from jax.experimental.pallas import tpu as pltpu
from jax.experimental.pallas import tpu_sc as plsc

E = 8
K = 2
NEG_INF = float("-inf")

M_BLK = 256
NT = 2048 * K // M_BLK + E
HB = 1536
NC, NS = 2, 16
NW = NC * NS


def _router_kernel(x_ref, gw_ref, i0_ref, i1_ref, g0_ref, g1_ref, aux_ref):
    x = x_ref[...]
    gw = gw_ref[...]
    logits = jax.lax.dot_general(
        x, gw, (((1,), (0,)), ((), ())), preferred_element_type=jnp.float32)
    T = logits.shape[0]
    ii = jax.lax.broadcasted_iota(jnp.int32, (T, E), 1)
    v0 = jnp.max(logits, axis=1, keepdims=True)
    i0 = jnp.min(jnp.where(logits == v0, ii, E), axis=1, keepdims=True)
    masked = jnp.where(ii == i0, NEG_INF, logits)
    v1 = jnp.max(masked, axis=1, keepdims=True)
    i1 = jnp.min(jnp.where(masked == v1, ii, E), axis=1, keepdims=True)
    s = jnp.exp(v1 - v0)
    g0 = 1.0 / (1.0 + s)
    g1 = s / (1.0 + s)
    i0_ref[...] = i0
    i1_ref[...] = i1
    g0_ref[...] = g0
    g1_ref[...] = g1
    gates = jnp.where(ii == i0, g0, 0.0) + jnp.where(ii == i1, g1, 0.0)
    m = jnp.max(logits, axis=1, keepdims=True)
    p = jnp.exp(logits - m)
    p = p / jnp.sum(p, axis=1, keepdims=True)
    f = jnp.mean(gates, axis=0, keepdims=True)
    P = jnp.mean(p, axis=0, keepdims=True)
    aux_ref[0, 0] = E * jnp.sum(f * P)


def _gmm_kernel(te_ref, na_ref, xs_ref, wg_ref, wu_ref, wd_ref,
                os_ref, acc_ref, wgb_ref, wub_ref, wdb_ref, key_ref):
    h = pl.program_id(0)
    t = pl.program_id(1)
    n_hb = pl.num_programs(0)

    @pl.when((h == 0) & (t == 0))
    def _():
        key_ref[0] = -1

    @pl.when(t < na_ref[0])
    def _():
        key = te_ref[t] * 256 + h

        @pl.when(key != key_ref[0])
        def _():
            wgb_ref[...] = wg_ref[0].astype(jnp.bfloat16)
            wub_ref[...] = wu_ref[0].astype(jnp.bfloat16)
            wdb_ref[...] = wd_ref[0].astype(jnp.bfloat16)
            key_ref[0] = key

        xb = xs_ref[0].astype(jnp.bfloat16)
        wgb = wgb_ref[...]
        wub = wub_ref[...]
        wdb = wdb_ref[...]
        g = jax.lax.dot_general(
            xb, wgb, (((1,), (0,)), ((), ())), preferred_element_type=jnp.float32)
        u = jax.lax.dot_general(
            xb, wub, (((1,), (0,)), ((), ())), preferred_element_type=jnp.float32)
        hid = ((g * jax.lax.logistic(g)) * u).astype(jnp.bfloat16)
        part = jax.lax.dot_general(
            hid, wdb, (((1,), (0,)), ((), ())), preferred_element_type=jnp.float32)

        @pl.when(h == 0)
        def _():
            acc_ref[t] = part.astype(jnp.bfloat16)

        @pl.when((h > 0) & (h < n_hb - 1))
        def _():
            acc_ref[t] += part.astype(jnp.bfloat16)

        @pl.when(h == n_hb - 1)
        def _():
            os_ref[0] = acc_ref[t].astype(jnp.float32) + part


def _make_dispatch(T, D, NTM):
    t_per_w = T // NW
    a_per_w = K * T // NW
    mesh = plsc.VectorSubcoreMesh(
        core_axis_name="c", subcore_axis_name="s", num_cores=NC, num_subcores=NS)

    @functools.partial(
        pl.kernel, mesh=mesh,
        out_type=jax.ShapeDtypeStruct((NTM, D), jnp.float32),
        scratch_types=[
            pltpu.VMEM((t_per_w,), jnp.int32),
            pltpu.VMEM((t_per_w, D), jnp.float32),
            pltpu.SemaphoreType.DMA,
        ],
    )
    def dispatch(x_hbm, pos0_hbm, pos1_hbm, xs_hbm, idx_v, rows_v, sem):
        wid = lax.axis_index("s") * NC + lax.axis_index("c")
        base = wid * t_per_w
        pltpu.sync_copy(x_hbm.at[pl.ds(base, t_per_w)], rows_v)
        pltpu.sync_copy(pos0_hbm.at[pl.ds(base, t_per_w)], idx_v)
        pltpu.async_copy(rows_v, xs_hbm.at[idx_v], sem).wait()
        pltpu.sync_copy(pos1_hbm.at[pl.ds(base, t_per_w)], idx_v)
        pltpu.async_copy(rows_v, xs_hbm.at[idx_v], sem).wait()

    return dispatch


def _make_combine(T, D, NTM):
    t_per_w = T // NW
    mesh = plsc.VectorSubcoreMesh(
        core_axis_name="c", subcore_axis_name="s", num_cores=NC, num_subcores=NS)

    @functools.partial(
        pl.kernel, mesh=mesh,
        out_type=[
            jax.ShapeDtypeStruct((T, D), jnp.float32),
            jax.ShapeDtypeStruct((T, D), jnp.float32),
        ],
        scratch_types=[
            pltpu.VMEM((t_per_w,), jnp.int32),
            pltpu.VMEM((t_per_w, D), jnp.float32),
            pltpu.SemaphoreType.DMA,
        ],
    )
    def combine(os_hbm, pos0_hbm, pos1_hbm, y0_hbm, y1_hbm, idx_v, rows_v, sem):
        wid = lax.axis_index("s") * NC + lax.axis_index("c")
        base = wid * t_per_w
        pltpu.sync_copy(pos0_hbm.at[pl.ds(base, t_per_w)], idx_v)
        pltpu.async_copy(os_hbm.at[idx_v], rows_v, sem).wait()
        pltpu.sync_copy(rows_v, y0_hbm.at[pl.ds(base, t_per_w)])
        pltpu.sync_copy(pos1_hbm.at[pl.ds(base, t_per_w)], idx_v)
        pltpu.async_copy(os_hbm.at[idx_v], rows_v, sem).wait()
        pltpu.sync_copy(rows_v, y1_hbm.at[pl.ds(base, t_per_w)])

    return combine


def _add_kernel(a_ref, b_ref, ga_ref, gb_ref, o_ref):
    o_ref[...] = a_ref[...] * ga_ref[...] + b_ref[...] * gb_ref[...]


@functools.partial(jax.jit, static_argnames=())
def kernel(x, gate_w, wg, wu, wd):
    B, S, D = x.shape
    H = wg.shape[2]
    T = B * S
    NTM = NT * M_BLK
    x_flat = x.reshape(T, D)

    i0, i1, g0, g1, aux = pl.pallas_call(
        _router_kernel,
        out_shape=(
            jax.ShapeDtypeStruct((T, 1), jnp.int32),
            jax.ShapeDtypeStruct((T, 1), jnp.int32),
            jax.ShapeDtypeStruct((T, 1), jnp.float32),
            jax.ShapeDtypeStruct((T, 1), jnp.float32),
            jax.ShapeDtypeStruct((1, 1), jnp.float32),
        ),
        in_specs=[
            pl.BlockSpec((T, D), lambda: (0, 0)),
            pl.BlockSpec((D, E), lambda: (0, 0)),
        ],
        out_specs=(
            pl.BlockSpec((T, 1), lambda: (0, 0)),
            pl.BlockSpec((T, 1), lambda: (0, 0)),
            pl.BlockSpec((T, 1), lambda: (0, 0)),
            pl.BlockSpec((T, 1), lambda: (0, 0)),
            pl.BlockSpec((1, 1), lambda: (0, 0), memory_space=pltpu.SMEM),
        ),
    )(x_flat, gate_w)
    aux_loss = aux[0, 0]

    e_iota = jnp.arange(E, dtype=jnp.int32)[None, :]
    oh0 = (i0 == e_iota).astype(jnp.int32)
    oh1 = (i1 == e_iota).astype(jnp.int32)
    both = oh0 + oh1
    counts = jnp.sum(both, axis=0)
    excl = jnp.cumsum(both, axis=0) - both
    rank0 = jnp.sum(excl * oh0, axis=1)
    rank1 = jnp.sum((excl + oh0) * oh1, axis=1)
    psize = ((counts + M_BLK - 1) // M_BLK) * M_BLK
    pstart = jnp.concatenate([jnp.zeros((1,), jnp.int32),
                              jnp.cumsum(psize)[:-1].astype(jnp.int32)])
    pos0 = jnp.sum(oh0 * pstart[None, :], axis=1) + rank0
    pos1 = jnp.sum(oh1 * pstart[None, :], axis=1) + rank1
    total = pstart[-1] + psize[-1]
    n_active = (total // M_BLK).astype(jnp.int32)[None]
    tile_start = jnp.arange(NT, dtype=jnp.int32) * M_BLK
    te = jnp.clip(jnp.sum(tile_start[:, None] >= pstart[None, :], axis=1) - 1,
                  0, E - 1).astype(jnp.int32)
    xs = _make_dispatch(T, D, NTM)(x_flat, pos0, pos1)

    n_hb = H // HB
    os_sorted = pl.pallas_call(
        _gmm_kernel,
        grid_spec=pltpu.PrefetchScalarGridSpec(
            num_scalar_prefetch=2,
            grid=(n_hb, NT),
            in_specs=[
                pl.BlockSpec((1, M_BLK, D), lambda h, t, te_r, na_r: (t, 0, 0)),
                pl.BlockSpec((1, D, HB), lambda h, t, te_r, na_r: (te_r[t], 0, h)),
                pl.BlockSpec((1, D, HB), lambda h, t, te_r, na_r: (te_r[t], 0, h)),
                pl.BlockSpec((1, HB, D), lambda h, t, te_r, na_r: (te_r[t], h, 0)),
            ],
            out_specs=pl.BlockSpec((1, M_BLK, D), lambda h, t, te_r, na_r: (t, 0, 0)),
            scratch_shapes=[pltpu.VMEM((NT, M_BLK, D), jnp.bfloat16),
                            pltpu.VMEM((D, HB), jnp.bfloat16),
                            pltpu.VMEM((D, HB), jnp.bfloat16),
                            pltpu.VMEM((HB, D), jnp.bfloat16),
                            pltpu.SMEM((1,), jnp.int32)],
        ),
        out_shape=jax.ShapeDtypeStruct((NT, M_BLK, D), jnp.float32),
    )(te, n_active, xs.reshape(NT, M_BLK, D), wg, wu, wd)

    y0, y1 = _make_combine(T, D, NTM)(
        os_sorted.reshape(NTM, D), pos0, pos1)

    out = pl.pallas_call(
        _add_kernel,
        out_shape=jax.ShapeDtypeStruct((T, D), jnp.float32),
        in_specs=[pl.BlockSpec((T, D), lambda: (0, 0)),
                  pl.BlockSpec((T, D), lambda: (0, 0)),
                  pl.BlockSpec((T, 1), lambda: (0, 0)),
                  pl.BlockSpec((T, 1), lambda: (0, 0))],
        out_specs=pl.BlockSpec((T, D), lambda: (0, 0)),
    )(y0, y1, g0, g1)

    return out.reshape(B, S, D), aux_loss


# speedup vs baseline: 1.1304x; 1.0374x over previous
import functools

import jax
import jax.numpy as jnp
from jax import lax
from jax.experimental import pallas as pl
from jax.experimental.pallas import tpu as pltpu
from jax.experimental.pallas import tpu_sc as plsc

E = 8
K = 2
NEG_INF = float("-inf")

M_BLK = 256
NT = 2048 * K // M_BLK + E
HB = 1536
NC, NS = 2, 16
NW = NC * NS


def _router_kernel(x_ref, gw_ref, pos0_ref, pos1_ref, g0_ref, g1_ref,
                   te_ref, na_ref, aux_ref):
    x = x_ref[...]
    gw = gw_ref[...]
    logits = jax.lax.dot_general(
        x, gw, (((1,), (0,)), ((), ())), preferred_element_type=jnp.float32)
    T = logits.shape[0]
    ii = jax.lax.broadcasted_iota(jnp.int32, (T, E), 1)
    v0 = jnp.max(logits, axis=1, keepdims=True)
    i0 = jnp.min(jnp.where(logits == v0, ii, E), axis=1, keepdims=True)
    masked = jnp.where(ii == i0, NEG_INF, logits)
    v1 = jnp.max(masked, axis=1, keepdims=True)
    i1 = jnp.min(jnp.where(masked == v1, ii, E), axis=1, keepdims=True)
    s = jnp.exp(v1 - v0)
    g0 = 1.0 / (1.0 + s)
    g1 = s / (1.0 + s)
    g0_ref[...] = g0
    g1_ref[...] = g1
    gates = jnp.where(ii == i0, g0, 0.0) + jnp.where(ii == i1, g1, 0.0)
    m = jnp.max(logits, axis=1, keepdims=True)
    p = jnp.exp(logits - m)
    p = p / jnp.sum(p, axis=1, keepdims=True)
    f = jnp.mean(gates, axis=0, keepdims=True)
    P = jnp.mean(p, axis=0, keepdims=True)
    aux_ref[0, 0] = E * jnp.sum(f * P)

    oh0 = jnp.where(ii == i0, 1.0, 0.0)
    oh1 = jnp.where(ii == i1, 1.0, 0.0)
    both = oh0 + oh1
    r_iota = jax.lax.broadcasted_iota(jnp.int32, (T, T), 0)
    c_iota = jax.lax.broadcasted_iota(jnp.int32, (T, T), 1)
    lt = jnp.where(r_iota > c_iota, 1.0, 0.0)
    excl = jax.lax.dot_general(
        lt, both, (((1,), (0,)), ((), ())), preferred_element_type=jnp.float32)
    counts = jnp.sum(both, axis=0, keepdims=True)
    rank0 = jnp.sum(excl * oh0, axis=1, keepdims=True)
    rank1 = jnp.sum((excl + oh0) * oh1, axis=1, keepdims=True)
    psize = jnp.floor((counts + (M_BLK - 1)) / M_BLK) * M_BLK
    e_row = jax.lax.broadcasted_iota(jnp.int32, (E, E), 0)
    e_col = jax.lax.broadcasted_iota(jnp.int32, (E, E), 1)
    lt8 = jnp.where(e_row < e_col, 1.0, 0.0)
    pstart = jax.lax.dot_general(
        psize, lt8, (((1,), (0,)), ((), ())),
        preferred_element_type=jnp.float32)
    pos0_ref[...] = (jnp.sum(oh0 * pstart, axis=1, keepdims=True)
                     + rank0).astype(jnp.int32)
    pos1_ref[...] = (jnp.sum(oh1 * pstart, axis=1, keepdims=True)
                     + rank1).astype(jnp.int32)
    total = jnp.sum(psize)
    na_ref[0, 0] = (total / M_BLK).astype(jnp.int32)
    ts = (jax.lax.broadcasted_iota(jnp.int32, (NT, E), 0) * M_BLK).astype(
        jnp.float32)
    te = jnp.sum(jnp.where(ts >= pstart, 1.0, 0.0), axis=1, keepdims=True) - 1.0
    te_ref[...] = jnp.clip(te, 0.0, E - 1.0).astype(jnp.int32)


def _gmm_kernel(te_ref, na_ref, xs_ref, wg_ref, wu_ref, wd_ref,
                os_ref, acc_ref):
    h = pl.program_id(0)
    t = pl.program_id(1)
    n_hb = pl.num_programs(0)

    @pl.when(t < na_ref[0])
    def _():
        xb = xs_ref[0].astype(jnp.bfloat16)
        wgb = wg_ref[0].astype(jnp.bfloat16)
        wub = wu_ref[0].astype(jnp.bfloat16)
        wdb = wd_ref[0].astype(jnp.bfloat16)
        g = jax.lax.dot_general(
            xb, wgb, (((1,), (0,)), ((), ())), preferred_element_type=jnp.float32)
        u = jax.lax.dot_general(
            xb, wub, (((1,), (0,)), ((), ())), preferred_element_type=jnp.float32)
        hid = ((g * jax.lax.logistic(g)) * u).astype(jnp.bfloat16)
        part = jax.lax.dot_general(
            hid, wdb, (((1,), (0,)), ((), ())), preferred_element_type=jnp.float32)

        @pl.when(h == 0)
        def _():
            acc_ref[t] = part

        @pl.when(h > 0)
        def _():
            acc_ref[t] += part

        @pl.when(h == n_hb - 1)
        def _():
            os_ref[0] = acc_ref[t]


def _make_dispatch(T, D, NTM):
    t_per_w = T // NW
    a_per_w = K * T // NW
    mesh = plsc.VectorSubcoreMesh(
        core_axis_name="c", subcore_axis_name="s", num_cores=NC, num_subcores=NS)

    @functools.partial(
        pl.kernel, mesh=mesh,
        out_type=jax.ShapeDtypeStruct((NTM, D), jnp.float32),
        scratch_types=[
            pltpu.VMEM((t_per_w,), jnp.int32),
            pltpu.VMEM((t_per_w, D), jnp.float32),
            pltpu.SemaphoreType.DMA,
        ],
    )
    def dispatch(x_hbm, pos0_hbm, pos1_hbm, xs_hbm, idx_v, rows_v, sem):
        wid = lax.axis_index("s") * NC + lax.axis_index("c")
        base = wid * t_per_w
        pltpu.sync_copy(x_hbm.at[pl.ds(base, t_per_w)], rows_v)
        pltpu.sync_copy(pos0_hbm.at[pl.ds(base, t_per_w)], idx_v)
        pltpu.async_copy(rows_v, xs_hbm.at[idx_v], sem).wait()
        pltpu.sync_copy(pos1_hbm.at[pl.ds(base, t_per_w)], idx_v)
        pltpu.async_copy(rows_v, xs_hbm.at[idx_v], sem).wait()

    return dispatch


def _make_combine(T, D, NTM):
    t_per_w = T // NW
    mesh = plsc.VectorSubcoreMesh(
        core_axis_name="c", subcore_axis_name="s", num_cores=NC, num_subcores=NS)

    @functools.partial(
        pl.kernel, mesh=mesh,
        out_type=[
            jax.ShapeDtypeStruct((T, D), jnp.float32),
            jax.ShapeDtypeStruct((T, D), jnp.float32),
        ],
        scratch_types=[
            pltpu.VMEM((t_per_w,), jnp.int32),
            pltpu.VMEM((t_per_w, D), jnp.float32),
            pltpu.SemaphoreType.DMA,
        ],
    )
    def combine(os_hbm, pos0_hbm, pos1_hbm, y0_hbm, y1_hbm, idx_v, rows_v, sem):
        wid = lax.axis_index("s") * NC + lax.axis_index("c")
        base = wid * t_per_w
        pltpu.sync_copy(pos0_hbm.at[pl.ds(base, t_per_w)], idx_v)
        pltpu.async_copy(os_hbm.at[idx_v], rows_v, sem).wait()
        pltpu.sync_copy(rows_v, y0_hbm.at[pl.ds(base, t_per_w)])
        pltpu.sync_copy(pos1_hbm.at[pl.ds(base, t_per_w)], idx_v)
        pltpu.async_copy(os_hbm.at[idx_v], rows_v, sem).wait()
        pltpu.sync_copy(rows_v, y1_hbm.at[pl.ds(base, t_per_w)])

    return combine


def _add_kernel(a_ref, b_ref, ga_ref, gb_ref, o_ref):
    o_ref[...] = a_ref[...] * ga_ref[...] + b_ref[...] * gb_ref[...]


@functools.partial(jax.jit, static_argnames=())
def kernel(x, gate_w, wg, wu, wd):
    B, S, D = x.shape
    H = wg.shape[2]
    T = B * S
    NTM = NT * M_BLK
    x_flat = x.reshape(T, D)

    pos0, pos1, g0, g1, te, na, aux = pl.pallas_call(
        _router_kernel,
        out_shape=(
            jax.ShapeDtypeStruct((T, 1), jnp.int32),
            jax.ShapeDtypeStruct((T, 1), jnp.int32),
            jax.ShapeDtypeStruct((T, 1), jnp.float32),
            jax.ShapeDtypeStruct((T, 1), jnp.float32),
            jax.ShapeDtypeStruct((NT, 1), jnp.int32),
            jax.ShapeDtypeStruct((1, 1), jnp.int32),
            jax.ShapeDtypeStruct((1, 1), jnp.float32),
        ),
        in_specs=[
            pl.BlockSpec((T, D), lambda: (0, 0)),
            pl.BlockSpec((D, E), lambda: (0, 0)),
        ],
        out_specs=(
            pl.BlockSpec((T, 1), lambda: (0, 0)),
            pl.BlockSpec((T, 1), lambda: (0, 0)),
            pl.BlockSpec((T, 1), lambda: (0, 0)),
            pl.BlockSpec((T, 1), lambda: (0, 0)),
            pl.BlockSpec((NT, 1), lambda: (0, 0)),
            pl.BlockSpec((1, 1), lambda: (0, 0), memory_space=pltpu.SMEM),
            pl.BlockSpec((1, 1), lambda: (0, 0), memory_space=pltpu.SMEM),
        ),
    )(x_flat, gate_w)
    aux_loss = aux[0, 0]
    te = te.reshape(NT)
    n_active = na.reshape(1)
    pos0 = pos0.reshape(T)
    pos1 = pos1.reshape(T)
    xs = _make_dispatch(T, D, NTM)(x_flat, pos0, pos1)

    n_hb = H // HB
    os_sorted = pl.pallas_call(
        _gmm_kernel,
        grid_spec=pltpu.PrefetchScalarGridSpec(
            num_scalar_prefetch=2,
            grid=(n_hb, NT),
            in_specs=[
                pl.BlockSpec((1, M_BLK, D), lambda h, t, te_r, na_r: (t, 0, 0)),
                pl.BlockSpec((1, D, HB), lambda h, t, te_r, na_r: (te_r[t], 0, h)),
                pl.BlockSpec((1, D, HB), lambda h, t, te_r, na_r: (te_r[t], 0, h)),
                pl.BlockSpec((1, HB, D), lambda h, t, te_r, na_r: (te_r[t], h, 0)),
            ],
            out_specs=pl.BlockSpec((1, M_BLK, D), lambda h, t, te_r, na_r: (t, 0, 0)),
            scratch_shapes=[pltpu.VMEM((NT, M_BLK, D), jnp.float32)],
        ),
        out_shape=jax.ShapeDtypeStruct((NT, M_BLK, D), jnp.float32),
    )(te, n_active, xs.reshape(NT, M_BLK, D), wg, wu, wd)

    y0, y1 = _make_combine(T, D, NTM)(
        os_sorted.reshape(NTM, D), pos0, pos1)

    out = pl.pallas_call(
        _add_kernel,
        out_shape=jax.ShapeDtypeStruct((T, D), jnp.float32),
        in_specs=[pl.BlockSpec((T, D), lambda: (0, 0)),
                  pl.BlockSpec((T, D), lambda: (0, 0)),
                  pl.BlockSpec((T, 1), lambda: (0, 0)),
                  pl.BlockSpec((T, 1), lambda: (0, 0))],
        out_specs=pl.BlockSpec((T, D), lambda: (0, 0)),
    )(y0, y1, g0, g1)

    return out.reshape(B, S, D), aux_loss


# speedup vs baseline: 1.1401x; 1.0085x over previous
import functools

import jax
import jax.numpy as jnp
from jax import lax
from jax.experimental import pallas as pl
from jax.experimental.pallas import tpu as pltpu
from jax.experimental.pallas import tpu_sc as plsc

E = 8
K = 2
NEG_INF = float("-inf")

M_BLK = 256
NT = 2048 * K // M_BLK + E
HB = 1536
NC, NS = 2, 16
NW = NC * NS


def _router_kernel(x_ref, gw_ref, pos0_ref, pos1_ref, g0_ref, g1_ref,
                   te_ref, na_ref, aux_ref):
    x = x_ref[...]
    gw = gw_ref[...]
    logits = jax.lax.dot_general(
        x, gw, (((1,), (0,)), ((), ())), preferred_element_type=jnp.float32)
    T = logits.shape[0]
    ii = jax.lax.broadcasted_iota(jnp.int32, (T, E), 1)
    v0 = jnp.max(logits, axis=1, keepdims=True)
    i0 = jnp.min(jnp.where(logits == v0, ii, E), axis=1, keepdims=True)
    masked = jnp.where(ii == i0, NEG_INF, logits)
    v1 = jnp.max(masked, axis=1, keepdims=True)
    i1 = jnp.min(jnp.where(masked == v1, ii, E), axis=1, keepdims=True)
    s = jnp.exp(v1 - v0)
    g0 = 1.0 / (1.0 + s)
    g1 = s / (1.0 + s)
    g0_ref[...] = g0
    g1_ref[...] = g1
    gates = jnp.where(ii == i0, g0, 0.0) + jnp.where(ii == i1, g1, 0.0)
    m = jnp.max(logits, axis=1, keepdims=True)
    p = jnp.exp(logits - m)
    p = p / jnp.sum(p, axis=1, keepdims=True)
    f = jnp.mean(gates, axis=0, keepdims=True)
    P = jnp.mean(p, axis=0, keepdims=True)
    aux_ref[0, 0] = E * jnp.sum(f * P)

    oh0 = jnp.where(ii == i0, 1.0, 0.0)
    oh1 = jnp.where(ii == i1, 1.0, 0.0)
    both = oh0 + oh1
    r_iota = jax.lax.broadcasted_iota(jnp.int32, (T, T), 0)
    c_iota = jax.lax.broadcasted_iota(jnp.int32, (T, T), 1)
    lt = jnp.where(r_iota > c_iota, 1.0, 0.0)
    excl = jax.lax.dot_general(
        lt, both, (((1,), (0,)), ((), ())), preferred_element_type=jnp.float32)
    counts = jnp.sum(both, axis=0, keepdims=True)
    rank0 = jnp.sum(excl * oh0, axis=1, keepdims=True)
    rank1 = jnp.sum((excl + oh0) * oh1, axis=1, keepdims=True)
    psize = jnp.floor((counts + (M_BLK - 1)) / M_BLK) * M_BLK
    e_row = jax.lax.broadcasted_iota(jnp.int32, (E, E), 0)
    e_col = jax.lax.broadcasted_iota(jnp.int32, (E, E), 1)
    lt8 = jnp.where(e_row < e_col, 1.0, 0.0)
    pstart = jax.lax.dot_general(
        psize, lt8, (((1,), (0,)), ((), ())),
        preferred_element_type=jnp.float32)
    pos0_ref[...] = (jnp.sum(oh0 * pstart, axis=1, keepdims=True)
                     + rank0).astype(jnp.int32)
    pos1_ref[...] = (jnp.sum(oh1 * pstart, axis=1, keepdims=True)
                     + rank1).astype(jnp.int32)
    total = jnp.sum(psize)
    na_ref[0, 0] = (total / M_BLK).astype(jnp.int32)
    ts = (jax.lax.broadcasted_iota(jnp.int32, (NT, E), 0) * M_BLK).astype(
        jnp.float32)
    te = jnp.sum(jnp.where(ts >= pstart, 1.0, 0.0), axis=1, keepdims=True) - 1.0
    te_ref[...] = jnp.clip(te, 0.0, E - 1.0).astype(jnp.int32)


def _gmm_kernel(te_ref, na_ref, xs_ref, wg_ref, wu_ref, wd_ref,
                os_ref, acc_ref):
    h = pl.program_id(0)
    t = pl.program_id(1)
    n_hb = pl.num_programs(0)

    @pl.when(t < na_ref[0])
    def _():
        xb = xs_ref[0].astype(jnp.bfloat16)
        wgb = wg_ref[0].astype(jnp.bfloat16)
        wub = wu_ref[0].astype(jnp.bfloat16)
        wdb = wd_ref[0].astype(jnp.bfloat16)
        g = jax.lax.dot_general(
            xb, wgb, (((1,), (0,)), ((), ())), preferred_element_type=jnp.float32)
        u = jax.lax.dot_general(
            xb, wub, (((1,), (0,)), ((), ())), preferred_element_type=jnp.float32)
        hid = ((g * jax.lax.logistic(g)) * u).astype(jnp.bfloat16)
        part = jax.lax.dot_general(
            hid, wdb, (((1,), (0,)), ((), ())), preferred_element_type=jnp.float32)

        @pl.when(h == 0)
        def _():
            acc_ref[t] = part

        @pl.when(h > 0)
        def _():
            acc_ref[t] += part

        @pl.when(h == n_hb - 1)
        def _():
            os_ref[0] = acc_ref[t]


def _make_dispatch(T, D, NTM):
    t_per_w = T // NW
    a_per_w = K * T // NW
    mesh = plsc.VectorSubcoreMesh(
        core_axis_name="c", subcore_axis_name="s", num_cores=NC, num_subcores=NS)

    @functools.partial(
        pl.kernel, mesh=mesh,
        out_type=jax.ShapeDtypeStruct((NTM, D), jnp.float32),
        scratch_types=[
            pltpu.VMEM((t_per_w,), jnp.int32),
            pltpu.VMEM((t_per_w,), jnp.int32),
            pltpu.VMEM((t_per_w, D), jnp.float32),
            pltpu.SemaphoreType.DMA,
        ],
    )
    def dispatch(x_hbm, pos0_hbm, pos1_hbm, xs_hbm, idx0_v, idx1_v, rows_v, sem):
        wid = lax.axis_index("s") * NC + lax.axis_index("c")
        base = wid * t_per_w
        c0 = pltpu.async_copy(x_hbm.at[pl.ds(base, t_per_w)], rows_v, sem)
        c1 = pltpu.async_copy(pos0_hbm.at[pl.ds(base, t_per_w)], idx0_v, sem)
        c2 = pltpu.async_copy(pos1_hbm.at[pl.ds(base, t_per_w)], idx1_v, sem)
        c0.wait()
        c1.wait()
        c2.wait()
        s0 = pltpu.async_copy(rows_v, xs_hbm.at[idx0_v], sem)
        s1 = pltpu.async_copy(rows_v, xs_hbm.at[idx1_v], sem)
        s0.wait()
        s1.wait()

    return dispatch


def _make_combine(T, D, NTM):
    t_per_w = T // NW
    mesh = plsc.VectorSubcoreMesh(
        core_axis_name="c", subcore_axis_name="s", num_cores=NC, num_subcores=NS)

    @functools.partial(
        pl.kernel, mesh=mesh,
        out_type=[
            jax.ShapeDtypeStruct((T, D), jnp.float32),
            jax.ShapeDtypeStruct((T, D), jnp.float32),
        ],
        scratch_types=[
            pltpu.VMEM((t_per_w,), jnp.int32),
            pltpu.VMEM((t_per_w,), jnp.int32),
            pltpu.VMEM((t_per_w, D), jnp.float32),
            pltpu.VMEM((t_per_w, D), jnp.float32),
            pltpu.SemaphoreType.DMA,
        ],
    )
    def combine(os_hbm, pos0_hbm, pos1_hbm, y0_hbm, y1_hbm,
                idx0_v, idx1_v, r0_v, r1_v, sem):
        wid = lax.axis_index("s") * NC + lax.axis_index("c")
        base = wid * t_per_w
        c0 = pltpu.async_copy(pos0_hbm.at[pl.ds(base, t_per_w)], idx0_v, sem)
        c1 = pltpu.async_copy(pos1_hbm.at[pl.ds(base, t_per_w)], idx1_v, sem)
        c0.wait()
        c1.wait()
        g0 = pltpu.async_copy(os_hbm.at[idx0_v], r0_v, sem)
        g1 = pltpu.async_copy(os_hbm.at[idx1_v], r1_v, sem)
        g0.wait()
        g1.wait()
        w0 = pltpu.async_copy(r0_v, y0_hbm.at[pl.ds(base, t_per_w)], sem)
        w1 = pltpu.async_copy(r1_v, y1_hbm.at[pl.ds(base, t_per_w)], sem)
        w0.wait()
        w1.wait()

    return combine


def _add_kernel(a_ref, b_ref, ga_ref, gb_ref, o_ref):
    o_ref[...] = a_ref[...] * ga_ref[...] + b_ref[...] * gb_ref[...]


@functools.partial(jax.jit, static_argnames=())
def kernel(x, gate_w, wg, wu, wd):
    B, S, D = x.shape
    H = wg.shape[2]
    T = B * S
    NTM = NT * M_BLK
    x_flat = x.reshape(T, D)

    pos0, pos1, g0, g1, te, na, aux = pl.pallas_call(
        _router_kernel,
        out_shape=(
            jax.ShapeDtypeStruct((T, 1), jnp.int32),
            jax.ShapeDtypeStruct((T, 1), jnp.int32),
            jax.ShapeDtypeStruct((T, 1), jnp.float32),
            jax.ShapeDtypeStruct((T, 1), jnp.float32),
            jax.ShapeDtypeStruct((NT, 1), jnp.int32),
            jax.ShapeDtypeStruct((1, 1), jnp.int32),
            jax.ShapeDtypeStruct((1, 1), jnp.float32),
        ),
        in_specs=[
            pl.BlockSpec((T, D), lambda: (0, 0)),
            pl.BlockSpec((D, E), lambda: (0, 0)),
        ],
        out_specs=(
            pl.BlockSpec((T, 1), lambda: (0, 0)),
            pl.BlockSpec((T, 1), lambda: (0, 0)),
            pl.BlockSpec((T, 1), lambda: (0, 0)),
            pl.BlockSpec((T, 1), lambda: (0, 0)),
            pl.BlockSpec((NT, 1), lambda: (0, 0)),
            pl.BlockSpec((1, 1), lambda: (0, 0), memory_space=pltpu.SMEM),
            pl.BlockSpec((1, 1), lambda: (0, 0), memory_space=pltpu.SMEM),
        ),
    )(x_flat, gate_w)
    aux_loss = aux[0, 0]
    te = te.reshape(NT)
    n_active = na.reshape(1)
    pos0 = pos0.reshape(T)
    pos1 = pos1.reshape(T)
    xs = _make_dispatch(T, D, NTM)(x_flat, pos0, pos1)

    n_hb = H // HB
    os_sorted = pl.pallas_call(
        _gmm_kernel,
        grid_spec=pltpu.PrefetchScalarGridSpec(
            num_scalar_prefetch=2,
            grid=(n_hb, NT),
            in_specs=[
                pl.BlockSpec((1, M_BLK, D), lambda h, t, te_r, na_r: (t, 0, 0)),
                pl.BlockSpec((1, D, HB), lambda h, t, te_r, na_r: (te_r[t], 0, h)),
                pl.BlockSpec((1, D, HB), lambda h, t, te_r, na_r: (te_r[t], 0, h)),
                pl.BlockSpec((1, HB, D), lambda h, t, te_r, na_r: (te_r[t], h, 0)),
            ],
            out_specs=pl.BlockSpec((1, M_BLK, D), lambda h, t, te_r, na_r: (t, 0, 0)),
            scratch_shapes=[pltpu.VMEM((NT, M_BLK, D), jnp.float32)],
        ),
        out_shape=jax.ShapeDtypeStruct((NT, M_BLK, D), jnp.float32),
    )(te, n_active, xs.reshape(NT, M_BLK, D), wg, wu, wd)

    y0, y1 = _make_combine(T, D, NTM)(
        os_sorted.reshape(NTM, D), pos0, pos1)

    out = pl.pallas_call(
        _add_kernel,
        out_shape=jax.ShapeDtypeStruct((T, D), jnp.float32),
        in_specs=[pl.BlockSpec((T, D), lambda: (0, 0)),
                  pl.BlockSpec((T, D), lambda: (0, 0)),
                  pl.BlockSpec((T, 1), lambda: (0, 0)),
                  pl.BlockSpec((T, 1), lambda: (0, 0))],
        out_specs=pl.BlockSpec((T, D), lambda: (0, 0)),
    )(y0, y1, g0, g1)

    return out.reshape(B, S, D), aux_loss


# speedup vs baseline: 1.1401x; 1.0000x over previous
import functools

import jax
import jax.numpy as jnp
from jax import lax
from jax.experimental import pallas as pl
from jax.experimental.pallas import tpu as pltpu
from jax.experimental.pallas import tpu_sc as plsc

E = 8
K = 2
NEG_INF = float("-inf")

M_BLK = 256
NT = 2048 * K // M_BLK + E
HB = 1536
NC, NS = 2, 16
NW = NC * NS


def _router_kernel(x_ref, gw_ref, pos0_ref, pos1_ref, g0_ref, g1_ref,
                   te_ref, na_ref, aux_ref):
    x = x_ref[...]
    gw = gw_ref[...]
    logits = jax.lax.dot_general(
        x, gw, (((1,), (0,)), ((), ())), preferred_element_type=jnp.float32)
    T = logits.shape[0]
    ii = jax.lax.broadcasted_iota(jnp.int32, (T, E), 1)
    v0 = jnp.max(logits, axis=1, keepdims=True)
    i0 = jnp.min(jnp.where(logits == v0, ii, E), axis=1, keepdims=True)
    masked = jnp.where(ii == i0, NEG_INF, logits)
    v1 = jnp.max(masked, axis=1, keepdims=True)
    i1 = jnp.min(jnp.where(masked == v1, ii, E), axis=1, keepdims=True)
    s = jnp.exp(v1 - v0)
    g0 = 1.0 / (1.0 + s)
    g1 = s / (1.0 + s)
    g0_ref[...] = g0
    g1_ref[...] = g1
    gates = jnp.where(ii == i0, g0, 0.0) + jnp.where(ii == i1, g1, 0.0)
    m = jnp.max(logits, axis=1, keepdims=True)
    p = jnp.exp(logits - m)
    p = p / jnp.sum(p, axis=1, keepdims=True)
    f = jnp.mean(gates, axis=0, keepdims=True)
    P = jnp.mean(p, axis=0, keepdims=True)
    aux_ref[0, 0] = E * jnp.sum(f * P)

    oh0 = jnp.where(ii == i0, 1.0, 0.0)
    oh1 = jnp.where(ii == i1, 1.0, 0.0)
    both = oh0 + oh1
    r_iota = jax.lax.broadcasted_iota(jnp.int32, (T, T), 0)
    c_iota = jax.lax.broadcasted_iota(jnp.int32, (T, T), 1)
    lt = jnp.where(r_iota > c_iota, 1.0, 0.0)
    excl = jax.lax.dot_general(
        lt, both, (((1,), (0,)), ((), ())), preferred_element_type=jnp.float32)
    counts = jnp.sum(both, axis=0, keepdims=True)
    rank0 = jnp.sum(excl * oh0, axis=1, keepdims=True)
    rank1 = jnp.sum((excl + oh0) * oh1, axis=1, keepdims=True)
    psize = jnp.floor((counts + (M_BLK - 1)) / M_BLK) * M_BLK
    e_row = jax.lax.broadcasted_iota(jnp.int32, (E, E), 0)
    e_col = jax.lax.broadcasted_iota(jnp.int32, (E, E), 1)
    lt8 = jnp.where(e_row < e_col, 1.0, 0.0)
    pstart = jax.lax.dot_general(
        psize, lt8, (((1,), (0,)), ((), ())),
        preferred_element_type=jnp.float32)
    pos0_ref[...] = (jnp.sum(oh0 * pstart, axis=1, keepdims=True)
                     + rank0).astype(jnp.int32)
    pos1_ref[...] = (jnp.sum(oh1 * pstart, axis=1, keepdims=True)
                     + rank1).astype(jnp.int32)
    total = jnp.sum(psize)
    na_ref[0, 0] = (total / M_BLK).astype(jnp.int32)
    ts = (jax.lax.broadcasted_iota(jnp.int32, (NT, E), 0) * M_BLK).astype(
        jnp.float32)
    te = jnp.sum(jnp.where(ts >= pstart, 1.0, 0.0), axis=1, keepdims=True) - 1.0
    te_ref[...] = jnp.clip(te, 0.0, E - 1.0).astype(jnp.int32)


def _gmm_kernel(te_ref, na_ref, xs_ref, wg_ref, wu_ref, wd_ref,
                os_ref, acc_ref):
    h = pl.program_id(0)
    t = pl.program_id(1)
    n_hb = pl.num_programs(0)

    @pl.when(t < na_ref[0])
    def _():
        xb = xs_ref[0].astype(jnp.bfloat16)
        wgb = wg_ref[0].astype(jnp.bfloat16)
        wub = wu_ref[0].astype(jnp.bfloat16)
        wdb = wd_ref[0].astype(jnp.bfloat16)
        g = jax.lax.dot_general(
            xb, wgb, (((1,), (0,)), ((), ())), preferred_element_type=jnp.float32)
        u = jax.lax.dot_general(
            xb, wub, (((1,), (0,)), ((), ())), preferred_element_type=jnp.float32)
        hid = ((g * jax.lax.logistic(g)) * u).astype(jnp.bfloat16)
        part = jax.lax.dot_general(
            hid, wdb, (((1,), (0,)), ((), ())), preferred_element_type=jnp.float32)

        @pl.when(h == 0)
        def _():
            acc_ref[t] = part

        @pl.when(h > 0)
        def _():
            acc_ref[t] += part

        @pl.when(h == n_hb - 1)
        def _():
            os_ref[0] = acc_ref[t]


def _make_dispatch(T, D, NTM):
    t_per_w = T // NW
    a_per_w = K * T // NW
    mesh = plsc.VectorSubcoreMesh(
        core_axis_name="c", subcore_axis_name="s", num_cores=NC, num_subcores=NS)

    @functools.partial(
        pl.kernel, mesh=mesh,
        out_type=jax.ShapeDtypeStruct((NTM, D), jnp.float32),
        scratch_types=[
            pltpu.VMEM((t_per_w,), jnp.int32),
            pltpu.VMEM((t_per_w,), jnp.int32),
            pltpu.VMEM((t_per_w, D), jnp.float32),
            pltpu.SemaphoreType.DMA,
        ],
    )
    def dispatch(x_hbm, pos0_hbm, pos1_hbm, xs_hbm, idx0_v, idx1_v, rows_v, sem):
        wid = lax.axis_index("s") * NC + lax.axis_index("c")
        base = wid * t_per_w
        c0 = pltpu.async_copy(x_hbm.at[pl.ds(base, t_per_w)], rows_v, sem)
        c1 = pltpu.async_copy(pos0_hbm.at[pl.ds(base, t_per_w)], idx0_v, sem)
        c2 = pltpu.async_copy(pos1_hbm.at[pl.ds(base, t_per_w)], idx1_v, sem)
        c0.wait()
        c1.wait()
        c2.wait()
        s0 = pltpu.async_copy(rows_v, xs_hbm.at[idx0_v], sem)
        s1 = pltpu.async_copy(rows_v, xs_hbm.at[idx1_v], sem)
        s0.wait()
        s1.wait()

    return dispatch


def _make_combine(T, D, NTM):
    t_per_w = T // NW
    mesh = plsc.VectorSubcoreMesh(
        core_axis_name="c", subcore_axis_name="s", num_cores=NC, num_subcores=NS)

    @functools.partial(
        pl.kernel, mesh=mesh,
        out_type=[
            jax.ShapeDtypeStruct((T, D), jnp.float32),
            jax.ShapeDtypeStruct((T, D), jnp.float32),
        ],
        scratch_types=[
            pltpu.VMEM((t_per_w,), jnp.int32),
            pltpu.VMEM((t_per_w,), jnp.int32),
            pltpu.VMEM((t_per_w, D), jnp.float32),
            pltpu.VMEM((t_per_w, D), jnp.float32),
            pltpu.SemaphoreType.DMA,
        ],
    )
    def combine(os_hbm, pos0_hbm, pos1_hbm, y0_hbm, y1_hbm,
                idx0_v, idx1_v, r0_v, r1_v, sem):
        wid = lax.axis_index("s") * NC + lax.axis_index("c")
        base = wid * t_per_w
        c0 = pltpu.async_copy(pos0_hbm.at[pl.ds(base, t_per_w)], idx0_v, sem)
        c1 = pltpu.async_copy(pos1_hbm.at[pl.ds(base, t_per_w)], idx1_v, sem)
        c0.wait()
        c1.wait()
        g0 = pltpu.async_copy(os_hbm.at[idx0_v], r0_v, sem)
        g1 = pltpu.async_copy(os_hbm.at[idx1_v], r1_v, sem)
        g0.wait()
        g1.wait()
        w0 = pltpu.async_copy(r0_v, y0_hbm.at[pl.ds(base, t_per_w)], sem)
        w1 = pltpu.async_copy(r1_v, y1_hbm.at[pl.ds(base, t_per_w)], sem)
        w0.wait()
        w1.wait()

    return combine


def _add_kernel(a_ref, b_ref, ga_ref, gb_ref, o_ref):
    o_ref[...] = a_ref[...] * ga_ref[...] + b_ref[...] * gb_ref[...]


@functools.partial(jax.jit, static_argnames=())
def kernel(x, gate_w, wg, wu, wd):
    B, S, D = x.shape
    H = wg.shape[2]
    T = B * S
    NTM = NT * M_BLK
    x_flat = x.reshape(T, D)

    pos0, pos1, g0, g1, te, na, aux = pl.pallas_call(
        _router_kernel,
        out_shape=(
            jax.ShapeDtypeStruct((T, 1), jnp.int32),
            jax.ShapeDtypeStruct((T, 1), jnp.int32),
            jax.ShapeDtypeStruct((T, 1), jnp.float32),
            jax.ShapeDtypeStruct((T, 1), jnp.float32),
            jax.ShapeDtypeStruct((NT, 1), jnp.int32),
            jax.ShapeDtypeStruct((1, 1), jnp.int32),
            jax.ShapeDtypeStruct((1, 1), jnp.float32),
        ),
        in_specs=[
            pl.BlockSpec((T, D), lambda: (0, 0)),
            pl.BlockSpec((D, E), lambda: (0, 0)),
        ],
        out_specs=(
            pl.BlockSpec((T, 1), lambda: (0, 0)),
            pl.BlockSpec((T, 1), lambda: (0, 0)),
            pl.BlockSpec((T, 1), lambda: (0, 0)),
            pl.BlockSpec((T, 1), lambda: (0, 0)),
            pl.BlockSpec((NT, 1), lambda: (0, 0)),
            pl.BlockSpec((1, 1), lambda: (0, 0), memory_space=pltpu.SMEM),
            pl.BlockSpec((1, 1), lambda: (0, 0), memory_space=pltpu.SMEM),
        ),
    )(x_flat, gate_w)
    aux_loss = aux[0, 0]
    te = te.reshape(NT)
    n_active = na.reshape(1)
    pos0 = pos0.reshape(T)
    pos1 = pos1.reshape(T)
    xs = _make_dispatch(T, D, NTM)(x_flat, pos0, pos1)

    n_hb = H // HB
    os_sorted = pl.pallas_call(
        _gmm_kernel,
        grid_spec=pltpu.PrefetchScalarGridSpec(
            num_scalar_prefetch=2,
            grid=(n_hb, NT),
            in_specs=[
                pl.BlockSpec((1, M_BLK, D), lambda h, t, te_r, na_r: (t, 0, 0)),
                pl.BlockSpec((1, D, HB), lambda h, t, te_r, na_r: (te_r[t], 0, h)),
                pl.BlockSpec((1, D, HB), lambda h, t, te_r, na_r: (te_r[t], 0, h)),
                pl.BlockSpec((1, HB, D), lambda h, t, te_r, na_r: (te_r[t], h, 0)),
            ],
            out_specs=pl.BlockSpec((1, M_BLK, D), lambda h, t, te_r, na_r: (t, 0, 0)),
            scratch_shapes=[pltpu.VMEM((NT, M_BLK, D), jnp.float32)],
        ),
        out_shape=jax.ShapeDtypeStruct((NT, M_BLK, D), jnp.float32),
        compiler_params=pltpu.CompilerParams(
            dimension_semantics=("arbitrary", "parallel")),
    )(te, n_active, xs.reshape(NT, M_BLK, D), wg, wu, wd)

    y0, y1 = _make_combine(T, D, NTM)(
        os_sorted.reshape(NTM, D), pos0, pos1)

    out = pl.pallas_call(
        _add_kernel,
        out_shape=jax.ShapeDtypeStruct((T, D), jnp.float32),
        in_specs=[pl.BlockSpec((T, D), lambda: (0, 0)),
                  pl.BlockSpec((T, D), lambda: (0, 0)),
                  pl.BlockSpec((T, 1), lambda: (0, 0)),
                  pl.BlockSpec((T, 1), lambda: (0, 0))],
        out_specs=pl.BlockSpec((T, D), lambda: (0, 0)),
    )(y0, y1, g0, g1)

    return out.reshape(B, S, D), aux_loss


# speedup vs baseline: 1.1409x; 1.0007x over previous
import functools

import jax
import jax.numpy as jnp
from jax import lax
from jax.experimental import pallas as pl
from jax.experimental.pallas import tpu as pltpu
from jax.experimental.pallas import tpu_sc as plsc

E = 8
K = 2
NEG_INF = float("-inf")

M_BLK = 256
NT = 2048 * K // M_BLK + E
HB = 1536
NC, NS = 2, 16
NW = NC * NS


def _router_kernel(x_ref, gw_ref, pos0_ref, pos1_ref, g0_ref, g1_ref,
                   te_ref, na_ref, aux_ref):
    x = x_ref[...]
    gw = gw_ref[...]
    logits = jax.lax.dot_general(
        x, gw, (((1,), (0,)), ((), ())), preferred_element_type=jnp.float32)
    T = logits.shape[0]
    ii = jax.lax.broadcasted_iota(jnp.int32, (T, E), 1)
    v0 = jnp.max(logits, axis=1, keepdims=True)
    i0 = jnp.min(jnp.where(logits == v0, ii, E), axis=1, keepdims=True)
    masked = jnp.where(ii == i0, NEG_INF, logits)
    v1 = jnp.max(masked, axis=1, keepdims=True)
    i1 = jnp.min(jnp.where(masked == v1, ii, E), axis=1, keepdims=True)
    s = jnp.exp(v1 - v0)
    g0 = 1.0 / (1.0 + s)
    g1 = s / (1.0 + s)
    g0_ref[...] = g0
    g1_ref[...] = g1
    gates = jnp.where(ii == i0, g0, 0.0) + jnp.where(ii == i1, g1, 0.0)
    m = jnp.max(logits, axis=1, keepdims=True)
    p = jnp.exp(logits - m)
    p = p / jnp.sum(p, axis=1, keepdims=True)
    f = jnp.mean(gates, axis=0, keepdims=True)
    P = jnp.mean(p, axis=0, keepdims=True)
    aux_ref[0, 0] = E * jnp.sum(f * P)

    oh0 = jnp.where(ii == i0, 1.0, 0.0)
    oh1 = jnp.where(ii == i1, 1.0, 0.0)
    both = oh0 + oh1
    r_iota = jax.lax.broadcasted_iota(jnp.int32, (T, T), 0)
    c_iota = jax.lax.broadcasted_iota(jnp.int32, (T, T), 1)
    lt = jnp.where(r_iota > c_iota, 1.0, 0.0)
    excl = jax.lax.dot_general(
        lt, both, (((1,), (0,)), ((), ())), preferred_element_type=jnp.float32)
    counts = jnp.sum(both, axis=0, keepdims=True)
    rank0 = jnp.sum(excl * oh0, axis=1, keepdims=True)
    rank1 = jnp.sum((excl + oh0) * oh1, axis=1, keepdims=True)
    psize = jnp.floor((counts + (M_BLK - 1)) / M_BLK) * M_BLK
    e_row = jax.lax.broadcasted_iota(jnp.int32, (E, E), 0)
    e_col = jax.lax.broadcasted_iota(jnp.int32, (E, E), 1)
    lt8 = jnp.where(e_row < e_col, 1.0, 0.0)
    pstart = jax.lax.dot_general(
        psize, lt8, (((1,), (0,)), ((), ())),
        preferred_element_type=jnp.float32)
    pos0_ref[...] = (jnp.sum(oh0 * pstart, axis=1, keepdims=True)
                     + rank0).astype(jnp.int32)
    pos1_ref[...] = (jnp.sum(oh1 * pstart, axis=1, keepdims=True)
                     + rank1).astype(jnp.int32)
    total = jnp.sum(psize)
    na_ref[0, 0] = (total / M_BLK).astype(jnp.int32)
    ts = (jax.lax.broadcasted_iota(jnp.int32, (NT, E), 0) * M_BLK).astype(
        jnp.float32)
    te = jnp.sum(jnp.where(ts >= pstart, 1.0, 0.0), axis=1, keepdims=True) - 1.0
    te_ref[...] = jnp.clip(te, 0.0, E - 1.0).astype(jnp.int32)


def _gmm_kernel(te_ref, na_ref, xs_ref, wg_ref, wu_ref, wd_ref,
                os_ref, acc_ref):
    h = pl.program_id(0)
    t = pl.program_id(1)
    n_hb = pl.num_programs(0)

    @pl.when(t < na_ref[0])
    def _():
        xb = xs_ref[0].astype(jnp.bfloat16)
        wgb = wg_ref[0].astype(jnp.bfloat16)
        wub = wu_ref[0].astype(jnp.bfloat16)
        wdb = wd_ref[0].astype(jnp.bfloat16)
        g = jax.lax.dot_general(
            xb, wgb, (((1,), (0,)), ((), ())), preferred_element_type=jnp.float32)
        u = jax.lax.dot_general(
            xb, wub, (((1,), (0,)), ((), ())), preferred_element_type=jnp.float32)
        hid = ((g * jax.lax.logistic(g)) * u).astype(jnp.bfloat16)
        part = jax.lax.dot_general(
            hid, wdb, (((1,), (0,)), ((), ())), preferred_element_type=jnp.float32)

        @pl.when(h == 0)
        def _():
            acc_ref[t] = part

        @pl.when(h > 0)
        def _():
            acc_ref[t] += part

        @pl.when(h == n_hb - 1)
        def _():
            os_ref[0] = acc_ref[t]


def _make_dispatch(T, D, NTM):
    t_per_w = T // NW
    mesh = plsc.VectorSubcoreMesh(
        core_axis_name="c", subcore_axis_name="s", num_cores=NC, num_subcores=NS)

    @functools.partial(
        pl.kernel, mesh=mesh,
        out_type=jax.ShapeDtypeStruct((NTM, D), jnp.float32),
        scratch_types=[
            pltpu.VMEM((t_per_w,), jnp.int32),
            pltpu.VMEM((t_per_w,), jnp.int32),
            pltpu.VMEM((t_per_w, D), jnp.float32),
            pltpu.SemaphoreType.DMA,
        ],
    )
    def dispatch(x_hbm, pos0_hbm, pos1_hbm, xs_hbm, idx0_v, idx1_v, rows_v, sem):
        wid = lax.axis_index("s") * NC + lax.axis_index("c")
        base = wid * t_per_w
        c0 = pltpu.async_copy(x_hbm.at[pl.ds(base, t_per_w)], rows_v, sem)
        c1 = pltpu.async_copy(pos0_hbm.at[pl.ds(base, t_per_w)], idx0_v, sem)
        c2 = pltpu.async_copy(pos1_hbm.at[pl.ds(base, t_per_w)], idx1_v, sem)
        c0.wait()
        c1.wait()
        c2.wait()
        s0 = pltpu.async_copy(rows_v, xs_hbm.at[idx0_v], sem)
        s1 = pltpu.async_copy(rows_v, xs_hbm.at[idx1_v], sem)
        s0.wait()
        s1.wait()

    return dispatch


def _make_combine(T, D, NTM):
    t_per_w = T // NW
    mesh = plsc.VectorSubcoreMesh(
        core_axis_name="c", subcore_axis_name="s", num_cores=NC, num_subcores=NS)

    @functools.partial(
        pl.kernel, mesh=mesh,
        out_type=[
            jax.ShapeDtypeStruct((T, D), jnp.float32),
            jax.ShapeDtypeStruct((T, D), jnp.float32),
        ],
        scratch_types=[
            pltpu.VMEM((t_per_w,), jnp.int32),
            pltpu.VMEM((t_per_w,), jnp.int32),
            pltpu.VMEM((t_per_w, D), jnp.float32),
            pltpu.VMEM((t_per_w, D), jnp.float32),
            pltpu.SemaphoreType.DMA,
        ],
    )
    def combine(os_hbm, pos0_hbm, pos1_hbm, y0_hbm, y1_hbm,
                idx0_v, idx1_v, r0_v, r1_v, sem):
        wid = lax.axis_index("s") * NC + lax.axis_index("c")
        base = wid * t_per_w
        c0 = pltpu.async_copy(pos0_hbm.at[pl.ds(base, t_per_w)], idx0_v, sem)
        c1 = pltpu.async_copy(pos1_hbm.at[pl.ds(base, t_per_w)], idx1_v, sem)
        c0.wait()
        c1.wait()
        g0 = pltpu.async_copy(os_hbm.at[idx0_v], r0_v, sem)
        g1 = pltpu.async_copy(os_hbm.at[idx1_v], r1_v, sem)
        g0.wait()
        g1.wait()
        w0 = pltpu.async_copy(r0_v, y0_hbm.at[pl.ds(base, t_per_w)], sem)
        w1 = pltpu.async_copy(r1_v, y1_hbm.at[pl.ds(base, t_per_w)], sem)
        w0.wait()
        w1.wait()

    return combine


def _add_kernel(a_ref, b_ref, ga_ref, gb_ref, o_ref):
    o_ref[...] = a_ref[...] * ga_ref[...] + b_ref[...] * gb_ref[...]


@functools.partial(jax.jit, static_argnames=())
def kernel(x, gate_w, wg, wu, wd):
    B, S, D = x.shape
    H = wg.shape[2]
    T = B * S
    NTM = NT * M_BLK
    x_flat = x.reshape(T, D)

    pos0, pos1, g0, g1, te, na, aux = pl.pallas_call(
        _router_kernel,
        out_shape=(
            jax.ShapeDtypeStruct((T, 1), jnp.int32),
            jax.ShapeDtypeStruct((T, 1), jnp.int32),
            jax.ShapeDtypeStruct((T, 1), jnp.float32),
            jax.ShapeDtypeStruct((T, 1), jnp.float32),
            jax.ShapeDtypeStruct((NT, 1), jnp.int32),
            jax.ShapeDtypeStruct((1, 1), jnp.int32),
            jax.ShapeDtypeStruct((1, 1), jnp.float32),
        ),
        in_specs=[
            pl.BlockSpec((T, D), lambda: (0, 0)),
            pl.BlockSpec((D, E), lambda: (0, 0)),
        ],
        out_specs=(
            pl.BlockSpec((T, 1), lambda: (0, 0)),
            pl.BlockSpec((T, 1), lambda: (0, 0)),
            pl.BlockSpec((T, 1), lambda: (0, 0)),
            pl.BlockSpec((T, 1), lambda: (0, 0)),
            pl.BlockSpec((NT, 1), lambda: (0, 0)),
            pl.BlockSpec((1, 1), lambda: (0, 0), memory_space=pltpu.SMEM),
            pl.BlockSpec((1, 1), lambda: (0, 0), memory_space=pltpu.SMEM),
        ),
    )(x_flat, gate_w)
    aux_loss = aux[0, 0]
    te = te.reshape(NT)
    n_active = na.reshape(1)
    pos0 = pos0.reshape(T)
    pos1 = pos1.reshape(T)
    xs = _make_dispatch(T, D, NTM)(x_flat, pos0, pos1)

    n_hb = H // HB
    os_sorted = pl.pallas_call(
        _gmm_kernel,
        grid_spec=pltpu.PrefetchScalarGridSpec(
            num_scalar_prefetch=2,
            grid=(n_hb, NT),
            in_specs=[
                pl.BlockSpec((1, M_BLK, D), lambda h, t, te_r, na_r: (t, 0, 0)),
                pl.BlockSpec((1, D, HB), lambda h, t, te_r, na_r: (te_r[t], 0, h)),
                pl.BlockSpec((1, D, HB), lambda h, t, te_r, na_r: (te_r[t], 0, h)),
                pl.BlockSpec((1, HB, D), lambda h, t, te_r, na_r: (te_r[t], h, 0)),
            ],
            out_specs=pl.BlockSpec((1, M_BLK, D), lambda h, t, te_r, na_r: (t, 0, 0)),
            scratch_shapes=[pltpu.VMEM((NT, M_BLK, D), jnp.float32)],
        ),
        out_shape=jax.ShapeDtypeStruct((NT, M_BLK, D), jnp.float32),
        compiler_params=pltpu.CompilerParams(
            dimension_semantics=("arbitrary", "parallel")),
    )(te, n_active, xs.reshape(NT, M_BLK, D), wg, wu, wd)

    y0, y1 = _make_combine(T, D, NTM)(
        os_sorted.reshape(NTM, D), pos0, pos1)

    out = pl.pallas_call(
        _add_kernel,
        out_shape=jax.ShapeDtypeStruct((T, D), jnp.float32),
        in_specs=[pl.BlockSpec((T, D), lambda: (0, 0)),
                  pl.BlockSpec((T, D), lambda: (0, 0)),
                  pl.BlockSpec((T, 1), lambda: (0, 0)),
                  pl.BlockSpec((T, 1), lambda: (0, 0))],
        out_specs=pl.BlockSpec((T, D), lambda: (0, 0)),
    )(y0, y1, g0, g1)

    return out.reshape(B, S, D), aux_loss
